# baseline (speedup 1.0000x reference)
import functools
import jax
import jax.numpy as jnp
from jax import lax
from jax.experimental import pallas as pl
from jax.experimental.pallas import tpu as pltpu
from jax.experimental.pallas import tpu_sc as plsc

_NC, _NS, _L = 2, 16, 16


def _sc_body(col_hbm, row_hbm, out_hbm, colbuf, rowbuf, chunk, sem):
    W = 32
    wid = lax.axis_index("s") * _NC + lax.axis_index("c")
    pltpu.sync_copy(col_hbm, colbuf)
    pltpu.sync_copy(row_hbm.at[wid], rowbuf)
    for i in range(W):
        for j in range(128 // _L):
            chunk[i, _L * j:_L * (j + 1)] = colbuf[i, _L * j:_L * (j + 1)]
    for j in range(128 // _L):
        v = rowbuf[_L * j:_L * (j + 1)]
        for i in range(W):
            chunk[i, 128 + _L * j:128 + _L * (j + 1)] = v
    copies = [
        pltpu.async_copy(chunk, out_hbm.at[b, pl.ds(wid * W, W), :], sem)
        for b in range(32)
    ]
    for c in copies:
        c.wait()


def kernel(x, h, w, row_embed, col_embed):
    B, HW, D = x.shape
    col = jax.lax.slice(col_embed, (0, 0), (32, 128))
    row = jax.lax.slice(row_embed, (0, 0), (32, 128))
    mesh = plsc.VectorSubcoreMesh(core_axis_name="c", subcore_axis_name="s")
    k = functools.partial(
        pl.kernel,
        mesh=mesh,
        out_type=jax.ShapeDtypeStruct((B, HW, D), jnp.float32),
        scratch_types=[
            pltpu.VMEM((32, 128), jnp.float32),
            pltpu.VMEM((128,), jnp.float32),
            pltpu.VMEM((32, 256), jnp.float32),
            pltpu.SemaphoreType.DMA,
        ],
    )(_sc_body)
    return k(col, row)
